# baseline (speedup 1.0000x reference)
import functools

import jax
import jax.numpy as jnp
from jax import lax
from jax.experimental import pallas as pl
from jax.experimental.pallas import tpu as pltpu
from jax.experimental.pallas import tpu_sc as plsc

PADDING_IDX = 0
NUM_CORES = 2
NUM_SUBCORES = 16
NUM_WORKERS = NUM_CORES * NUM_SUBCORES


@functools.partial(jax.jit, static_argnums=(2,))
def _sc_gather(idx_flat, table_pairs, dims):
    batch, seq, D = dims
    B = idx_flat.shape[0]
    rows_per_w = batch // NUM_WORKERS
    b_per_w = B // NUM_WORKERS

    mesh = plsc.VectorSubcoreMesh(core_axis_name="c", subcore_axis_name="s")

    @functools.partial(
        pl.kernel,
        mesh=mesh,
        out_type=jax.ShapeDtypeStruct((batch, seq, 2 * D), jnp.float32),
        scratch_types=[
            pltpu.VMEM((b_per_w,), jnp.int32),
            pltpu.VMEM((4, seq, 2 * D), jnp.float32),
            pltpu.SemaphoreType.DMA,
            pltpu.SemaphoreType.DMA,
            pltpu.SemaphoreType.DMA,
            pltpu.SemaphoreType.DMA,
            pltpu.SemaphoreType.DMA,
            pltpu.SemaphoreType.DMA,
            pltpu.SemaphoreType.DMA,
            pltpu.SemaphoreType.DMA,
        ],
        compiler_params=pltpu.CompilerParams(use_tc_tiling_on_sc=True),
    )
    def k(idx_hbm, table_hbm, out_hbm, idx_v, rows_v,
          g0, g1, g2, g3, w0, w1, w2, w3):
        wid = lax.axis_index("s") * NUM_CORES + lax.axis_index("c")
        row0 = wid * rows_per_w
        gsem = (g0, g1, g2, g3)
        wsem = (w0, w1, w2, w3)

        pltpu.sync_copy(idx_hbm.at[pl.ds(row0 * seq, b_per_w)], idx_v)

        gathers = [None] * 4
        writes = [None] * 4
        for i in range(rows_per_w + 2):
            b = i % 4
            if i < rows_per_w:
                if i >= 4:
                    writes[b].wait()
                gathers[b] = pltpu.async_copy(
                    table_hbm.at[idx_v.at[pl.ds(i * seq, seq)]],
                    rows_v.at[b],
                    gsem[b],
                )
            if i >= 2:
                j = i - 2
                bj = j % 4
                gathers[bj].wait()
                writes[bj] = pltpu.async_copy(
                    rows_v.at[bj],
                    out_hbm.at[row0 + j],
                    wsem[bj],
                )
        for t in range(min(4, rows_per_w)):
            writes[(rows_per_w - 1 - t) % 4].wait()

    return k(idx_flat, table_pairs)


def _mask_body(x_ref, o_ref):
    o_ref[...] = (x_ref[...] != PADDING_IDX).astype(jnp.float32)


def kernel(x, table):
    batch, seq = x.shape
    vocab, D = table.shape
    idx_flat = x.reshape(batch * seq)

    table_pad = jnp.pad(table, ((0, 0), (0, D)))
    out_p = _sc_gather(idx_flat, table_pad, (batch, seq, D))
    out_emb = out_p[:, :, :D]

    grid = 8
    rows_per_block = batch // grid
    mask = pl.pallas_call(
        _mask_body,
        out_shape=jax.ShapeDtypeStruct((batch, seq), jnp.float32),
        grid=(grid,),
        in_specs=[pl.BlockSpec((rows_per_block, seq), lambda i: (i, 0))],
        out_specs=pl.BlockSpec((rows_per_block, seq), lambda i: (i, 0)),
    )(x)

    return (out_emb, mask)


# speedup vs baseline: 1.0659x; 1.0659x over previous
import functools

import jax
import jax.numpy as jnp
from jax import lax
from jax.experimental import pallas as pl
from jax.experimental.pallas import tpu as pltpu
from jax.experimental.pallas import tpu_sc as plsc

PADDING_IDX = 0
NUM_CORES = 2
NUM_SUBCORES = 16
NUM_WORKERS = NUM_CORES * NUM_SUBCORES


@functools.partial(jax.jit, static_argnums=(2,))
def _sc_gather(idx_flat, table_pairs, dims):
    batch, seq, D = dims
    B = idx_flat.shape[0]
    rows_per_w = batch // NUM_WORKERS
    b_per_w = B // NUM_WORKERS

    mesh = plsc.VectorSubcoreMesh(core_axis_name="c", subcore_axis_name="s")

    @functools.partial(
        pl.kernel,
        mesh=mesh,
        out_type=jax.ShapeDtypeStruct((batch, seq, 2 * D), jnp.float32),
        scratch_types=[
            pltpu.VMEM((b_per_w,), jnp.int32),
            pltpu.VMEM((4, seq, 2 * D), jnp.float32),
            pltpu.SemaphoreType.DMA,
            pltpu.SemaphoreType.DMA,
            pltpu.SemaphoreType.DMA,
            pltpu.SemaphoreType.DMA,
            pltpu.SemaphoreType.DMA,
            pltpu.SemaphoreType.DMA,
            pltpu.SemaphoreType.DMA,
            pltpu.SemaphoreType.DMA,
        ],
        compiler_params=pltpu.CompilerParams(use_tc_tiling_on_sc=True),
    )
    def k(idx_hbm, table_hbm, out_hbm, idx_v, rows_v,
          g0, g1, g2, g3, w0, w1, w2, w3):
        wid = lax.axis_index("s") * NUM_CORES + lax.axis_index("c")
        row0 = wid * rows_per_w
        gsem = (g0, g1, g2, g3)
        wsem = (w0, w1, w2, w3)

        pltpu.sync_copy(idx_hbm.at[pl.ds(row0 * seq, b_per_w)], idx_v)

        gathers = [None] * 4
        writes = [None] * 4
        for i in range(rows_per_w + 2):
            b = i % 4
            if i < rows_per_w:
                if i >= 4:
                    writes[b].wait()
                gathers[b] = pltpu.async_copy(
                    table_hbm.at[idx_v.at[pl.ds(i * seq, seq)]],
                    rows_v.at[b],
                    gsem[b],
                )
            if i >= 2:
                j = i - 2
                bj = j % 4
                gathers[bj].wait()
                writes[bj] = pltpu.async_copy(
                    rows_v.at[bj],
                    out_hbm.at[row0 + j],
                    wsem[bj],
                )
        for t in range(min(4, rows_per_w)):
            writes[(rows_per_w - 1 - t) % 4].wait()

    return k(idx_flat, table_pairs)


def _mask_body(x_ref, o_ref):
    o_ref[...] = (x_ref[...] != PADDING_IDX).astype(jnp.float32)


def _tp_body(tt_ref, o_ref):
    t = tt_ref[...].T
    o_ref[...] = jnp.concatenate([t, jnp.zeros_like(t)], axis=1)


def kernel(x, table):
    batch, seq = x.shape
    vocab, D = table.shape
    idx_flat = x.reshape(batch * seq)

    VB = 2048
    table_t = table.T
    table_pad = pl.pallas_call(
        _tp_body,
        out_shape=jax.ShapeDtypeStruct((vocab, 2 * D), jnp.float32),
        grid=(pl.cdiv(vocab, VB),),
        in_specs=[pl.BlockSpec((D, VB), lambda i: (0, i))],
        out_specs=pl.BlockSpec((VB, 2 * D), lambda i: (i, 0)),
    )(table_t)
    out_p = _sc_gather(idx_flat, table_pad, (batch, seq, D))
    out_emb = out_p[:, :, :D]

    grid = 8
    rows_per_block = batch // grid
    mask = pl.pallas_call(
        _mask_body,
        out_shape=jax.ShapeDtypeStruct((batch, seq), jnp.float32),
        grid=(grid,),
        in_specs=[pl.BlockSpec((rows_per_block, seq), lambda i: (i, 0))],
        out_specs=pl.BlockSpec((rows_per_block, seq), lambda i: (i, 0)),
    )(x)

    return (out_emb, mask)


# speedup vs baseline: 1.3421x; 1.2592x over previous
import functools

import jax
import jax.numpy as jnp
from jax import lax
from jax.experimental import pallas as pl
from jax.experimental.pallas import tpu as pltpu
from jax.experimental.pallas import tpu_sc as plsc

PADDING_IDX = 0
NUM_CORES = 2
NUM_SUBCORES = 16
NUM_WORKERS = NUM_CORES * NUM_SUBCORES


@functools.partial(jax.jit, static_argnums=(2,))
def _sc_gather(idx_flat, table_pairs, dims):
    batch, seq, D = dims
    B = idx_flat.shape[0]
    rows_per_w = batch // NUM_WORKERS
    b_per_w = B // NUM_WORKERS

    mesh = plsc.VectorSubcoreMesh(core_axis_name="c", subcore_axis_name="s")

    @functools.partial(
        pl.kernel,
        mesh=mesh,
        out_type=jax.ShapeDtypeStruct((batch, seq, 2 * D), jnp.float32),
        scratch_types=[
            pltpu.VMEM((b_per_w,), jnp.int32),
            pltpu.VMEM((4, seq, 2 * D), jnp.float32),
            pltpu.SemaphoreType.DMA,
            pltpu.SemaphoreType.DMA,
            pltpu.SemaphoreType.DMA,
            pltpu.SemaphoreType.DMA,
            pltpu.SemaphoreType.DMA,
            pltpu.SemaphoreType.DMA,
            pltpu.SemaphoreType.DMA,
            pltpu.SemaphoreType.DMA,
        ],
        compiler_params=pltpu.CompilerParams(use_tc_tiling_on_sc=True),
    )
    def k(idx_hbm, table_hbm, out_hbm, idx_v, rows_v,
          g0, g1, g2, g3, w0, w1, w2, w3):
        wid = lax.axis_index("s") * NUM_CORES + lax.axis_index("c")
        row0 = wid * rows_per_w
        gsem = (g0, g1, g2, g3)
        wsem = (w0, w1, w2, w3)

        pltpu.sync_copy(idx_hbm.at[pl.ds(row0 * seq, b_per_w)], idx_v)

        gathers = [None] * 4
        writes = [None] * 4
        for i in range(rows_per_w + 2):
            b = i % 4
            if i < rows_per_w:
                if i >= 4:
                    writes[b].wait()
                gathers[b] = pltpu.async_copy(
                    table_hbm.at[idx_v.at[pl.ds(i * seq, seq)]],
                    rows_v.at[b],
                    gsem[b],
                )
            if i >= 2:
                j = i - 2
                bj = j % 4
                gathers[bj].wait()
                writes[bj] = pltpu.async_copy(
                    rows_v.at[bj],
                    out_hbm.at[row0 + j],
                    wsem[bj],
                )
        for t in range(min(4, rows_per_w)):
            writes[(rows_per_w - 1 - t) % 4].wait()

    return k(idx_flat, table_pairs)


def _mask_body(x_ref, o_ref):
    o_ref[...] = (x_ref[...] != PADDING_IDX).astype(jnp.float32)


def _tp_body(tt_ref, o_ref):
    t = tt_ref[...].T
    o_ref[...] = jnp.concatenate([t, jnp.zeros_like(t)], axis=1)


def kernel(x, table):
    batch, seq = x.shape
    vocab, D = table.shape
    idx_flat = x.reshape(batch * seq)

    VB = 8192
    table_t = table.T
    table_pad = pl.pallas_call(
        _tp_body,
        out_shape=jax.ShapeDtypeStruct((vocab, 2 * D), jnp.float32),
        grid=(pl.cdiv(vocab, VB),),
        in_specs=[pl.BlockSpec((D, VB), lambda i: (0, i))],
        out_specs=pl.BlockSpec((VB, 2 * D), lambda i: (i, 0)),
    )(table_t)
    out_p = _sc_gather(idx_flat, table_pad, (batch, seq, D))
    out_emb = out_p[:, :, :D]

    grid = 8
    rows_per_block = batch // grid
    mask = pl.pallas_call(
        _mask_body,
        out_shape=jax.ShapeDtypeStruct((batch, seq), jnp.float32),
        grid=(grid,),
        in_specs=[pl.BlockSpec((rows_per_block, seq), lambda i: (i, 0))],
        out_specs=pl.BlockSpec((rows_per_block, seq), lambda i: (i, 0)),
    )(x)

    return (out_emb, mask)


# speedup vs baseline: 1.3785x; 1.0271x over previous
import functools

import jax
import jax.numpy as jnp
from jax import lax
from jax.experimental import pallas as pl
from jax.experimental.pallas import tpu as pltpu
from jax.experimental.pallas import tpu_sc as plsc

PADDING_IDX = 0
NUM_CORES = 2
NUM_SUBCORES = 16
NUM_WORKERS = NUM_CORES * NUM_SUBCORES


@functools.partial(jax.jit, static_argnums=(2,))
def _sc_gather(idx_flat, table_pairs, dims):
    batch, seq, D = dims
    B = idx_flat.shape[0]
    rows_per_w = batch // NUM_WORKERS
    b_per_w = B // NUM_WORKERS

    mesh = plsc.VectorSubcoreMesh(core_axis_name="c", subcore_axis_name="s")

    @functools.partial(
        pl.kernel,
        mesh=mesh,
        out_type=jax.ShapeDtypeStruct((batch, seq, 2 * D), jnp.float32),
        scratch_types=[
            pltpu.VMEM((b_per_w,), jnp.int32),
            pltpu.VMEM((4, seq, 2 * D), jnp.float32),
            pltpu.SemaphoreType.DMA,
            pltpu.SemaphoreType.DMA,
            pltpu.SemaphoreType.DMA,
            pltpu.SemaphoreType.DMA,
            pltpu.SemaphoreType.DMA,
            pltpu.SemaphoreType.DMA,
            pltpu.SemaphoreType.DMA,
            pltpu.SemaphoreType.DMA,
        ],
        compiler_params=pltpu.CompilerParams(use_tc_tiling_on_sc=True),
    )
    def k(idx_hbm, table_hbm, out_hbm, idx_v, rows_v,
          g0, g1, g2, g3, w0, w1, w2, w3):
        wid = lax.axis_index("s") * NUM_CORES + lax.axis_index("c")
        row0 = wid * rows_per_w
        gsem = (g0, g1, g2, g3)
        wsem = (w0, w1, w2, w3)

        pltpu.sync_copy(idx_hbm.at[pl.ds(row0 * seq, b_per_w)], idx_v)

        gathers = [None] * 4
        writes = [None] * 4
        for i in range(rows_per_w + 2):
            b = i % 4
            if i < rows_per_w:
                if i >= 4:
                    writes[b].wait()
                gathers[b] = pltpu.async_copy(
                    table_hbm.at[idx_v.at[pl.ds(i * seq, seq)]],
                    rows_v.at[b],
                    gsem[b],
                )
            if i >= 2:
                j = i - 2
                bj = j % 4
                gathers[bj].wait()
                writes[bj] = pltpu.async_copy(
                    rows_v.at[bj],
                    out_hbm.at[row0 + j],
                    wsem[bj],
                )
        for t in range(min(4, rows_per_w)):
            writes[(rows_per_w - 1 - t) % 4].wait()

    return k(idx_flat, table_pairs)


def _mask_body(x_ref, o_ref):
    o_ref[...] = (x_ref[...] != PADDING_IDX).astype(jnp.float32)


def _tp_body(tt_ref, o_ref):
    t = tt_ref[...].T
    o_ref[...] = jnp.concatenate([t, jnp.zeros_like(t)], axis=1)


def kernel(x, table):
    batch, seq = x.shape
    vocab, D = table.shape
    idx_flat = x.reshape(batch * seq)

    VB = 16384
    table_t = table.T
    table_pad = pl.pallas_call(
        _tp_body,
        out_shape=jax.ShapeDtypeStruct((vocab, 2 * D), jnp.float32),
        grid=(pl.cdiv(vocab, VB),),
        in_specs=[pl.BlockSpec((D, VB), lambda i: (0, i))],
        out_specs=pl.BlockSpec((VB, 2 * D), lambda i: (i, 0)),
    )(table_t)
    out_p = _sc_gather(idx_flat, table_pad, (batch, seq, D))
    out_emb = out_p[:, :, :D]

    grid = 8
    rows_per_block = batch // grid
    mask = pl.pallas_call(
        _mask_body,
        out_shape=jax.ShapeDtypeStruct((batch, seq), jnp.float32),
        grid=(grid,),
        in_specs=[pl.BlockSpec((rows_per_block, seq), lambda i: (i, 0))],
        out_specs=pl.BlockSpec((rows_per_block, seq), lambda i: (i, 0)),
    )(x)

    return (out_emb, mask)


# speedup vs baseline: 1.3911x; 1.0092x over previous
import functools

import jax
import jax.numpy as jnp
from jax import lax
from jax.experimental import pallas as pl
from jax.experimental.pallas import tpu as pltpu
from jax.experimental.pallas import tpu_sc as plsc

PADDING_IDX = 0
NUM_CORES = 2
NUM_SUBCORES = 16
NUM_WORKERS = NUM_CORES * NUM_SUBCORES


@functools.partial(jax.jit, static_argnums=(2,))
def _sc_gather(idx_flat, table_pairs, dims):
    batch, seq, D = dims
    B = idx_flat.shape[0]
    rows_per_w = batch // NUM_WORKERS
    b_per_w = B // NUM_WORKERS

    mesh = plsc.VectorSubcoreMesh(core_axis_name="c", subcore_axis_name="s")

    @functools.partial(
        pl.kernel,
        mesh=mesh,
        out_type=jax.ShapeDtypeStruct((batch, seq, 2 * D), jnp.float32),
        scratch_types=[
            pltpu.VMEM((b_per_w,), jnp.int32),
            pltpu.VMEM((4, seq, 2 * D), jnp.float32),
            pltpu.SemaphoreType.DMA,
            pltpu.SemaphoreType.DMA,
            pltpu.SemaphoreType.DMA,
            pltpu.SemaphoreType.DMA,
            pltpu.SemaphoreType.DMA,
            pltpu.SemaphoreType.DMA,
            pltpu.SemaphoreType.DMA,
            pltpu.SemaphoreType.DMA,
        ],
        compiler_params=pltpu.CompilerParams(use_tc_tiling_on_sc=True),
    )
    def k(idx_hbm, table_hbm, out_hbm, idx_v, rows_v,
          g0, g1, g2, g3, w0, w1, w2, w3):
        wid = lax.axis_index("s") * NUM_CORES + lax.axis_index("c")
        row0 = wid * rows_per_w
        gsem = (g0, g1, g2, g3)
        wsem = (w0, w1, w2, w3)

        pltpu.sync_copy(idx_hbm.at[pl.ds(row0 * seq, b_per_w)], idx_v)

        gathers = [None] * 4
        writes = [None] * 4
        for i in range(rows_per_w + 2):
            b = i % 4
            if i < rows_per_w:
                if i >= 4:
                    writes[b].wait()
                gathers[b] = pltpu.async_copy(
                    table_hbm.at[idx_v.at[pl.ds(i * seq, seq)]],
                    rows_v.at[b],
                    gsem[b],
                )
            if i >= 2:
                j = i - 2
                bj = j % 4
                gathers[bj].wait()
                writes[bj] = pltpu.async_copy(
                    rows_v.at[bj],
                    out_hbm.at[row0 + j],
                    wsem[bj],
                )
        for t in range(min(4, rows_per_w)):
            writes[(rows_per_w - 1 - t) % 4].wait()

    return k(idx_flat, table_pairs)


def _mask_body(x_ref, o_ref):
    o_ref[...] = (x_ref[...] != PADDING_IDX).astype(jnp.float32)


def _tp_body(tt_ref, o_ref):
    t = tt_ref[...].T
    o_ref[...] = jnp.concatenate([t, jnp.zeros_like(t)], axis=1)


def kernel(x, table):
    batch, seq = x.shape
    vocab, D = table.shape
    idx_flat = x.reshape(batch * seq)

    VB = 32768
    table_t = table.T
    table_pad = pl.pallas_call(
        _tp_body,
        out_shape=jax.ShapeDtypeStruct((vocab, 2 * D), jnp.float32),
        grid=(pl.cdiv(vocab, VB),),
        in_specs=[pl.BlockSpec((D, VB), lambda i: (0, i))],
        out_specs=pl.BlockSpec((VB, 2 * D), lambda i: (i, 0)),
    )(table_t)
    out_p = _sc_gather(idx_flat, table_pad, (batch, seq, D))
    out_emb = out_p[:, :, :D]

    grid = 8
    rows_per_block = batch // grid
    mask = pl.pallas_call(
        _mask_body,
        out_shape=jax.ShapeDtypeStruct((batch, seq), jnp.float32),
        grid=(grid,),
        in_specs=[pl.BlockSpec((rows_per_block, seq), lambda i: (i, 0))],
        out_specs=pl.BlockSpec((rows_per_block, seq), lambda i: (i, 0)),
    )(x)

    return (out_emb, mask)
